# Initial kernel scaffold; baseline (speedup 1.0000x reference)
import jax
import jax.numpy as jnp
from jax.experimental import pallas as pl


def kernel(queries, keys):
    raise NotImplementedError("write your pallas kernel here")


# speedup vs baseline: 3.0227x; 3.0227x over previous
import functools

import jax
import jax.numpy as jnp
from jax.experimental import pallas as pl
from jax.experimental.pallas import tpu as pltpu

Q = 1024
D = 32
K = 100000
TOPK = 8
LANES = 128
BK = 512
KPAD = 100352
NSTEPS = KPAD // BK
SUB = BK // LANES
NEG = float("-inf")
BIGI = 2**30


def _topk_body(q_ref, k_ref, vals_ref, idx_ref, qn_ref, rv_ref, ri_ref):
    j = pl.program_id(0)

    @pl.when(j == 0)
    def _init():
        q = q_ref[...]
        qn = q / (jnp.sqrt(jnp.sum(q * q, axis=-1, keepdims=True)) + 1e-9)
        qn_ref[...] = qn
        rv_ref[...] = jnp.full((TOPK, Q, LANES), NEG, jnp.float32)
        ri_ref[...] = jnp.zeros((TOPK, Q, LANES), jnp.int32)

    kb = k_ref[...]
    kn = kb / (jnp.sqrt(jnp.sum(kb * kb, axis=-1, keepdims=True)) + 1e-9)
    sims = jax.lax.dot_general(
        qn_ref[...], kn,
        (((1,), (1,)), ((), ())),
        preferred_element_type=jnp.float32,
        precision=jax.lax.Precision.DEFAULT,
    )
    base = j * BK
    colid = base + jax.lax.broadcasted_iota(jnp.int32, (Q, BK), 1)
    sims = jnp.where(colid < K, sims, NEG)

    for t in range(SUB):
        nv = sims[:, t * LANES:(t + 1) * LANES]
        ni = colid[:, t * LANES:(t + 1) * LANES]
        for i in range(TOPK):
            rv_i = rv_ref[i]
            ri_i = ri_ref[i]
            cond = nv > rv_i
            rv_ref[i] = jnp.where(cond, nv, rv_i)
            ri_ref[i] = jnp.where(cond, ni, ri_i)
            nv = jnp.where(cond, rv_i, nv)
            ni = jnp.where(cond, ri_i, ni)

    @pl.when(j == NSTEPS - 1)
    def _final():
        rv = [rv_ref[i] for i in range(TOPK)]
        ri = [ri_ref[i] for i in range(TOPK)]
        out_v = []
        out_i = []
        for _ in range(TOPK):
            m = rv[0]
            for i in range(1, TOPK):
                m = jnp.maximum(m, rv[i])
            mrow = jnp.max(m, axis=1, keepdims=True)
            ci = jnp.full((Q, LANES), BIGI, jnp.int32)
            for i in range(TOPK):
                ci = jnp.minimum(ci, jnp.where(rv[i] == mrow, ri[i], BIGI))
            widx = jnp.min(ci, axis=1, keepdims=True)
            out_v.append(mrow)
            out_i.append(widx)
            for i in range(TOPK):
                kill = (ri[i] == widx) & (rv[i] == mrow)
                rv[i] = jnp.where(kill, NEG, rv[i])
        vals_ref[...] = jnp.concatenate(out_v, axis=1)
        idx_ref[...] = jnp.concatenate(out_i, axis=1)


@jax.jit
def _run(queries, keys_padded):
    return pl.pallas_call(
        _topk_body,
        grid=(NSTEPS,),
        in_specs=[
            pl.BlockSpec((Q, D), lambda j: (0, 0)),
            pl.BlockSpec((BK, D), lambda j: (j, 0)),
        ],
        out_specs=[
            pl.BlockSpec((Q, TOPK), lambda j: (0, 0)),
            pl.BlockSpec((Q, TOPK), lambda j: (0, 0)),
        ],
        out_shape=[
            jax.ShapeDtypeStruct((Q, TOPK), jnp.float32),
            jax.ShapeDtypeStruct((Q, TOPK), jnp.int32),
        ],
        scratch_shapes=[
            pltpu.VMEM((Q, D), jnp.float32),
            pltpu.VMEM((TOPK, Q, LANES), jnp.float32),
            pltpu.VMEM((TOPK, Q, LANES), jnp.int32),
        ],
        compiler_params=pltpu.CompilerParams(
            dimension_semantics=("arbitrary",),
        ),
    )(queries, keys_padded)


def kernel(queries, keys):
    keys_padded = jnp.pad(keys, ((0, KPAD - K), (0, 0)))
    vals, idx = _run(queries, keys_padded)
    return vals, idx
